# Initial kernel scaffold; baseline (speedup 1.0000x reference)
import jax
import jax.numpy as jnp
from jax.experimental import pallas as pl


def kernel(x, x1, edge_index, edge_index1, gcn1_W, gcn1_b, gcn2_W, gcn2_b, sage1_Wn, sage1_b, sage1_Ws, sage2_Wn, sage2_b, sage2_Ws, gat1_W, gat1_Wb, gat1_a, gat1_ab, gat2_W, gat2_Wb, gat2_a, gat2_ab, mlp_W1, mlp_b1, ln_g, ln_b, mlp_W2, mlp_b2):
    raise NotImplementedError("write your pallas kernel here")


# speedup vs baseline: 1.3415x; 1.3415x over previous
import jax
import jax.numpy as jnp
from jax.experimental import pallas as pl

N = 10000
E = 320000


def _seg_sum(vals, idx, n):
    return jax.ops.segment_sum(vals, idx, num_segments=n)


def _mean_body(x_ref, o_ref):
    o_ref[...] = jnp.sum(x_ref[...]).reshape(1, 1)


def kernel(x, x1, edge_index, edge_index1, gcn1_W, gcn1_b, gcn2_W, gcn2_b,
           sage1_Wn, sage1_b, sage1_Ws, sage2_Wn, sage2_b, sage2_Ws,
           gat1_W, gat1_Wb, gat1_a, gat1_ab, gat2_W, gat2_Wb, gat2_a, gat2_ab,
           mlp_W1, mlp_b1, ln_g, ln_b, mlp_W2, mlp_b2):
    ei = edge_index.astype(jnp.int32)
    ei1 = edge_index1.astype(jnp.int32)
    s0, d0 = ei[0], ei[1]
    s1, d1 = ei1[0], ei1[1]

    ones = jnp.ones((E,), jnp.float32)
    indeg0 = _seg_sum(ones, d0, N)
    indeg1 = _seg_sum(ones, d1, N)

    def gcn(xin, src, dst, indeg, W, b):
        dinv = 1.0 / jnp.sqrt(indeg + 1.0)
        xs = xin * dinv[:, None]
        agg = _seg_sum(xs[src], dst, N)
        out = (dinv[:, None] * (agg + xs)) @ W + b
        return out

    xa = gcn(x, s0, d0, indeg0, gcn1_W, gcn1_b)
    xb = gcn(x1, s1, d1, indeg1, gcn2_W, gcn2_b)
    x0 = jax.nn.relu(xa) + jax.nn.relu(xb)

    def sage(xin, src, dst, indeg, Wn, bn, Ws):
        p = xin @ Wn
        agg = _seg_sum(p[src], dst, N) / jnp.maximum(indeg, 1.0)[:, None]
        return agg + bn + xin @ Ws

    xc_s = sage(x0, s0, d0, indeg0, sage1_Wn, sage1_b, sage1_Ws)

    def gat(xin, src, dst, W, Wb, a, ab):
        h = xin @ W + Wb
        hd = h @ a[: h.shape[1], 0]
        hs = h @ a[h.shape[1]:, 0]
        e = hd[dst] + hs[src] + ab[0]
        e = jnp.where(e > 0, e, 0.2 * e)
        m = jax.ops.segment_max(e, dst, num_segments=N)
        m = jnp.where(jnp.isfinite(m), m, 0.0)
        ex = jnp.exp(e - m[dst])
        denom = _seg_sum(ex, dst, N)
        num = _seg_sum(ex[:, None] * h[src], dst, N)
        return num / jnp.where(denom > 0, denom, 1.0)[:, None]

    xc = jax.nn.relu(gat(xc_s, s0, d0, gat1_W, gat1_Wb, gat1_a, gat1_ab))

    def mlp(z):
        z = z @ mlp_W1 + mlp_b1
        mu = jnp.mean(z, axis=-1, keepdims=True)
        var = jnp.var(z, axis=-1, keepdims=True)
        z = (z - mu) / jnp.sqrt(var + 1e-5) * ln_g + ln_b
        return z @ mlp_W2 + mlp_b2

    xc = mlp(xc)

    x1_0b = jax.nn.relu(xb) + xb
    xd_s = sage(x1_0b, s1, d1, indeg1, sage2_Wn, sage2_b, sage2_Ws)
    xd = jax.nn.relu(gat(xd_s, s1, d1, gat2_W, gat2_Wb, gat2_a, gat2_ab))
    xd = mlp(xd)

    cat = jnp.concatenate([xc, xd], axis=0)
    total = pl.pallas_call(
        _mean_body,
        out_shape=jax.ShapeDtypeStruct((1, 1), jnp.float32),
    )(cat)
    return (total / (cat.shape[0] * cat.shape[1])).reshape(1, 1)


# speedup vs baseline: 2.2638x; 1.6875x over previous
import functools

import jax
import jax.numpy as jnp
from jax import lax
from jax.experimental import pallas as pl
from jax.experimental.pallas import tpu as pltpu
from jax.experimental.pallas import tpu_sc as plsc

N = 10000
E = 320000
NC = 2
NS = 16
NW = NC * NS
EPW = E // NW
CH = 1000
RCH = 256
RNCH = E // RCH
RJMAX = -(-RNCH // NW)
NPAD = 10240
SPT = NPAD // NS

_mesh = plsc.VectorSubcoreMesh(core_axis_name="c", subcore_axis_name="s")


def _wid():
    return lax.axis_index("s") * NC + lax.axis_index("c")


def _deg_body(dst0, dst1, ones_hbm, zeros_hbm, out, acc0, acc1, ones_v,
              idx0, idx1, sem):
    cid = lax.axis_index("c")
    sid = lax.axis_index("s")
    wid = _wid()
    pltpu.sync_copy(zeros_hbm.at[pl.ds(0, SPT)], acc0.at[pl.ds(sid * SPT, SPT)])
    pltpu.sync_copy(zeros_hbm.at[pl.ds(0, SPT)], acc1.at[pl.ds(sid * SPT, SPT)])
    pltpu.sync_copy(ones_hbm.at[pl.ds(0, CH)], ones_v)
    plsc.subcore_barrier()

    def it(k, carry):
        base = wid * EPW + k * CH
        pltpu.sync_copy(dst0.at[pl.ds(base, CH)], idx0)
        pltpu.sync_copy(dst1.at[pl.ds(base, CH)], idx1)
        pltpu.sync_copy(ones_v, acc0.at[idx0], add=True)
        pltpu.sync_copy(ones_v, acc1.at[idx1], add=True)
        return carry

    lax.fori_loop(0, EPW // CH, it, 0)
    plsc.subcore_barrier()
    pltpu.sync_copy(acc0.at[pl.ds(sid * SPT, SPT)],
                    out.at[pl.ds(cid * NPAD + sid * SPT, SPT)])
    pltpu.sync_copy(acc1.at[pl.ds(sid * SPT, SPT)],
                    out.at[pl.ds((NC + cid) * NPAD + sid * SPT, SPT)])


_deg_kernel = pl.kernel(
    _deg_body,
    out_type=jax.ShapeDtypeStruct((2 * NC * NPAD,), jnp.float32),
    mesh=_mesh,
    scratch_types=[
        pltpu.VMEM_SHARED((NPAD,), jnp.float32),
        pltpu.VMEM_SHARED((NPAD,), jnp.float32),
        pltpu.VMEM((CH,), jnp.float32),
        pltpu.VMEM((CH,), jnp.int32),
        pltpu.VMEM((CH,), jnp.int32),
        pltpu.SemaphoreType.DMA,
    ],
)


def _rowagg_body(table, src, dst, zeros_hbm, out, acc, idx_s, idx_d, rows,
                 sem):
    cid = lax.axis_index("c")
    sid = lax.axis_index("s")
    wid = _wid()
    pltpu.sync_copy(zeros_hbm.at[pl.ds(0, SPT)],
                    acc.at[pl.ds(sid * SPT, SPT)])
    plsc.subcore_barrier()

    def it(j, carry):
        chunk = j * NW + wid

        @pl.when(chunk < RNCH)
        def _():
            base = chunk * RCH
            pltpu.sync_copy(src.at[pl.ds(base, RCH)], idx_s)
            pltpu.sync_copy(dst.at[pl.ds(base, RCH)], idx_d)
            pltpu.async_copy(table.at[idx_s], rows, sem).wait()
            pltpu.sync_copy(rows, acc.at[idx_d], add=True)

        return carry

    lax.fori_loop(0, RJMAX, it, 0)
    plsc.subcore_barrier()
    pltpu.sync_copy(acc.at[pl.ds(sid * SPT, SPT)],
                    out.at[cid, pl.ds(sid * SPT, SPT)])


def _make_rowagg(D):
    return pl.kernel(
        _rowagg_body,
        out_type=jax.ShapeDtypeStruct((NC, NPAD, D), jnp.float32),
        mesh=_mesh,
        scratch_types=[
            pltpu.VMEM_SHARED((NPAD, D), jnp.float32),
            pltpu.VMEM((RCH,), jnp.int32),
            pltpu.VMEM((RCH,), jnp.int32),
            pltpu.VMEM((RCH, D), jnp.float32),
            pltpu.SemaphoreType.DMA,
        ],
    )


_rowagg128 = _make_rowagg(128)

_ZROWS = None


def _mean_body(x_ref, o_ref):
    o_ref[...] = jnp.sum(x_ref[...]).reshape(1, 1)


def kernel(x, x1, edge_index, edge_index1, gcn1_W, gcn1_b, gcn2_W, gcn2_b,
           sage1_Wn, sage1_b, sage1_Ws, sage2_Wn, sage2_b, sage2_Ws,
           gat1_W, gat1_Wb, gat1_a, gat1_ab, gat2_W, gat2_Wb, gat2_a, gat2_ab,
           mlp_W1, mlp_b1, ln_g, ln_b, mlp_W2, mlp_b2):
    ei = edge_index.astype(jnp.int32)
    ei1 = edge_index1.astype(jnp.int32)
    s0, d0 = ei[0], ei[1]
    s1, d1 = ei1[0], ei1[1]

    ones_ch = jnp.ones((CH,), jnp.float32)
    zeros_1d = jnp.zeros((SPT,), jnp.float32)
    zeros_rows = jnp.zeros((SPT, 128), jnp.float32)

    deg = _deg_kernel(d0, d1, ones_ch, zeros_1d).reshape(2, NC, NPAD)
    indeg0 = deg[0, 0, :N] + deg[0, 1, :N]
    indeg1 = deg[1, 0, :N] + deg[1, 1, :N]

    def gcn(xin, src, dst, indeg, W, b):
        dinv = 1.0 / jnp.sqrt(indeg + 1.0)
        xs = xin * dinv[:, None]
        parts = _rowagg128(xs, src, dst, zeros_rows)
        agg = parts[0, :N] + parts[1, :N]
        return (dinv[:, None] * (agg + xs)) @ W + b

    xa = gcn(x, s0, d0, indeg0, gcn1_W, gcn1_b)
    xb = gcn(x1, s1, d1, indeg1, gcn2_W, gcn2_b)
    x0 = jax.nn.relu(xa) + jax.nn.relu(xb)

    def sage(xin, src, dst, indeg, Wn, bn, Ws):
        p = xin @ Wn
        parts = _rowagg128(p, src, dst, zeros_rows)
        agg = (parts[0, :N] + parts[1, :N]) / jnp.maximum(indeg, 1.0)[:, None]
        return agg + bn + xin @ Ws

    xc_s = sage(x0, s0, d0, indeg0, sage1_Wn, sage1_b, sage1_Ws)

    def gat(xin, src, dst, W, Wb, a, ab):
        h = xin @ W + Wb
        hd = h @ a[: h.shape[1], 0]
        hs = h @ a[h.shape[1]:, 0]
        e = hd[dst] + hs[src] + ab[0]
        e = jnp.where(e > 0, e, 0.2 * e)
        ex = jnp.exp(e)
        denom = jax.ops.segment_sum(ex, dst, num_segments=N)
        num = jax.ops.segment_sum(ex[:, None] * h[src], dst, num_segments=N)
        return num / jnp.where(denom > 0, denom, 1.0)[:, None]

    xc = jax.nn.relu(gat(xc_s, s0, d0, gat1_W, gat1_Wb, gat1_a, gat1_ab))

    def mlp(z):
        z = z @ mlp_W1 + mlp_b1
        mu = jnp.mean(z, axis=-1, keepdims=True)
        var = jnp.var(z, axis=-1, keepdims=True)
        z = (z - mu) / jnp.sqrt(var + 1e-5) * ln_g + ln_b
        return z @ mlp_W2 + mlp_b2

    xc = mlp(xc)

    x1_0b = jax.nn.relu(xb) + xb
    xd_s = sage(x1_0b, s1, d1, indeg1, sage2_Wn, sage2_b, sage2_Ws)
    xd = jax.nn.relu(gat(xd_s, s1, d1, gat2_W, gat2_Wb, gat2_a, gat2_ab))
    xd = mlp(xd)

    cat = jnp.concatenate([xc, xd], axis=0)
    total = pl.pallas_call(
        _mean_body,
        out_shape=jax.ShapeDtypeStruct((1, 1), jnp.float32),
    )(cat)
    return (total / (cat.shape[0] * cat.shape[1])).reshape(1, 1)


# speedup vs baseline: 19.3686x; 8.5558x over previous
import functools

import jax
import jax.numpy as jnp
from jax import lax
from jax.experimental import pallas as pl
from jax.experimental.pallas import tpu as pltpu
from jax.experimental.pallas import tpu_sc as plsc

N = 10000
E = 320000
NC = 2
NS = 16
NW = NC * NS
EPW = E // NW
CH = 1000
RCH = 256
RNCH = E // RCH
RJMAX = -(-RNCH // NW)
NPAD = 10240
SPT = NPAD // NS

_mesh = plsc.VectorSubcoreMesh(core_axis_name="c", subcore_axis_name="s")


def _wid():
    return lax.axis_index("s") * NC + lax.axis_index("c")


def _deg_body(dst0, dst1, ones_hbm, zeros_hbm, out, acc0, acc1, ones_v,
              idx0, idx1, sem):
    cid = lax.axis_index("c")
    sid = lax.axis_index("s")
    wid = _wid()
    pltpu.sync_copy(zeros_hbm.at[pl.ds(0, SPT)], acc0.at[pl.ds(sid * SPT, SPT)])
    pltpu.sync_copy(zeros_hbm.at[pl.ds(0, SPT)], acc1.at[pl.ds(sid * SPT, SPT)])
    pltpu.sync_copy(ones_hbm.at[pl.ds(0, CH)], ones_v)
    plsc.subcore_barrier()

    def it(k, carry):
        base = wid * EPW + k * CH
        pltpu.sync_copy(dst0.at[pl.ds(base, CH)], idx0)
        pltpu.sync_copy(dst1.at[pl.ds(base, CH)], idx1)
        pltpu.sync_copy(ones_v, acc0.at[idx0], add=True)
        pltpu.sync_copy(ones_v, acc1.at[idx1], add=True)
        return carry

    lax.fori_loop(0, EPW // CH, it, 0)
    plsc.subcore_barrier()
    pltpu.sync_copy(acc0.at[pl.ds(sid * SPT, SPT)],
                    out.at[pl.ds(cid * NPAD + sid * SPT, SPT)])
    pltpu.sync_copy(acc1.at[pl.ds(sid * SPT, SPT)],
                    out.at[pl.ds((NC + cid) * NPAD + sid * SPT, SPT)])


_deg_kernel = pl.kernel(
    _deg_body,
    out_type=jax.ShapeDtypeStruct((2 * NC * NPAD,), jnp.float32),
    mesh=_mesh,
    scratch_types=[
        pltpu.VMEM_SHARED((NPAD,), jnp.float32),
        pltpu.VMEM_SHARED((NPAD,), jnp.float32),
        pltpu.VMEM((CH,), jnp.float32),
        pltpu.VMEM((CH,), jnp.int32),
        pltpu.VMEM((CH,), jnp.int32),
        pltpu.SemaphoreType.DMA,
    ],
)


def _rowagg_body(table, src, dst, zeros_hbm, out, acc, idx_s, idx_d, rows,
                 sem):
    cid = lax.axis_index("c")
    sid = lax.axis_index("s")
    wid = _wid()
    pltpu.sync_copy(zeros_hbm.at[pl.ds(0, SPT)],
                    acc.at[pl.ds(sid * SPT, SPT)])
    plsc.subcore_barrier()

    def it(j, carry):
        chunk = j * NW + wid

        @pl.when(chunk < RNCH)
        def _():
            base = chunk * RCH
            pltpu.sync_copy(src.at[pl.ds(base, RCH)], idx_s)
            pltpu.sync_copy(dst.at[pl.ds(base, RCH)], idx_d)
            pltpu.async_copy(table.at[idx_s], rows, sem).wait()
            pltpu.sync_copy(rows, acc.at[idx_d], add=True)

        return carry

    lax.fori_loop(0, RJMAX, it, 0)
    plsc.subcore_barrier()
    pltpu.sync_copy(acc.at[pl.ds(sid * SPT, SPT)],
                    out.at[cid, pl.ds(sid * SPT, SPT)])


def _make_rowagg(D):
    return pl.kernel(
        _rowagg_body,
        out_type=jax.ShapeDtypeStruct((NC, NPAD, D), jnp.float32),
        mesh=_mesh,
        scratch_types=[
            pltpu.VMEM_SHARED((NPAD, D), jnp.float32),
            pltpu.VMEM((RCH,), jnp.int32),
            pltpu.VMEM((RCH,), jnp.int32),
            pltpu.VMEM((RCH, D), jnp.float32),
            pltpu.SemaphoreType.DMA,
        ],
    )


_rowagg128 = _make_rowagg(128)

GCH = 256
GNCH = E // GCH
GJMAX = -(-GNCH // NW)


def _gat_body(h, ed, es, src, dst, zrows, z1d, outn, outd,
              accn, accd, idx_s, idx_d, rows, edv, esv, exv, sem):
    cid = lax.axis_index("c")
    sid = lax.axis_index("s")
    wid = _wid()
    pltpu.sync_copy(zrows.at[pl.ds(0, SPT)], accn.at[pl.ds(sid * SPT, SPT)])
    pltpu.sync_copy(z1d.at[pl.ds(0, SPT)], accd.at[pl.ds(sid * SPT, SPT)])
    plsc.subcore_barrier()

    def it(j, carry):
        chunk = j * NW + wid

        @pl.when(chunk < GNCH)
        def _():
            base = chunk * GCH
            pltpu.sync_copy(src.at[pl.ds(base, GCH)], idx_s)
            pltpu.sync_copy(dst.at[pl.ds(base, GCH)], idx_d)
            pltpu.async_copy(es.at[idx_s], esv, sem).wait()
            pltpu.async_copy(ed.at[idx_d], edv, sem).wait()
            pltpu.async_copy(h.at[idx_s], rows, sem).wait()

            for g in range(GCH // 16):
                e = edv[pl.ds(g * 16, 16)] + esv[pl.ds(g * 16, 16)]
                ex = jnp.exp(jnp.maximum(e, 0.2 * e))
                exv[pl.ds(g * 16, 16)] = ex
                for jj in range(16):
                    i = g * 16 + jj
                    b = jnp.full((16,), ex[jj])
                    for q in range(4):
                        rows[i, pl.ds(q * 16, 16)] = (
                            rows[i, pl.ds(q * 16, 16)] * b)

            pltpu.sync_copy(exv, accd.at[idx_d], add=True)
            pltpu.sync_copy(rows, accn.at[idx_d], add=True)

        return carry

    lax.fori_loop(0, GJMAX, it, 0)
    plsc.subcore_barrier()
    pltpu.sync_copy(accn.at[pl.ds(sid * SPT, SPT)],
                    outn.at[cid, pl.ds(sid * SPT, SPT)])
    pltpu.sync_copy(accd.at[pl.ds(sid * SPT, SPT)],
                    outd.at[pl.ds(cid * NPAD + sid * SPT, SPT)])


_gat_kernel = pl.kernel(
    _gat_body,
    out_type=(jax.ShapeDtypeStruct((NC, NPAD, 128), jnp.float32),
              jax.ShapeDtypeStruct((NC * NPAD,), jnp.float32)),
    mesh=_mesh,
    scratch_types=[
        pltpu.VMEM_SHARED((NPAD, 128), jnp.float32),
        pltpu.VMEM_SHARED((NPAD,), jnp.float32),
        pltpu.VMEM((GCH,), jnp.int32),
        pltpu.VMEM((GCH,), jnp.int32),
        pltpu.VMEM((GCH, 128), jnp.float32),
        pltpu.VMEM((GCH,), jnp.float32),
        pltpu.VMEM((GCH,), jnp.float32),
        pltpu.VMEM((GCH,), jnp.float32),
        pltpu.SemaphoreType.DMA,
    ],
)

_ZROWS = None


def _mean_body(x_ref, o_ref):
    o_ref[...] = jnp.sum(x_ref[...]).reshape(1, 1)


def kernel(x, x1, edge_index, edge_index1, gcn1_W, gcn1_b, gcn2_W, gcn2_b,
           sage1_Wn, sage1_b, sage1_Ws, sage2_Wn, sage2_b, sage2_Ws,
           gat1_W, gat1_Wb, gat1_a, gat1_ab, gat2_W, gat2_Wb, gat2_a, gat2_ab,
           mlp_W1, mlp_b1, ln_g, ln_b, mlp_W2, mlp_b2):
    ei = edge_index.astype(jnp.int32)
    ei1 = edge_index1.astype(jnp.int32)
    s0, d0 = ei[0], ei[1]
    s1, d1 = ei1[0], ei1[1]

    ones_ch = jnp.ones((CH,), jnp.float32)
    zeros_1d = jnp.zeros((SPT,), jnp.float32)
    zeros_rows = jnp.zeros((SPT, 128), jnp.float32)

    deg = _deg_kernel(d0, d1, ones_ch, zeros_1d).reshape(2, NC, NPAD)
    indeg0 = deg[0, 0, :N] + deg[0, 1, :N]
    indeg1 = deg[1, 0, :N] + deg[1, 1, :N]

    def gcn(xin, src, dst, indeg, W, b):
        dinv = 1.0 / jnp.sqrt(indeg + 1.0)
        xs = xin * dinv[:, None]
        parts = _rowagg128(xs, src, dst, zeros_rows)
        agg = parts[0, :N] + parts[1, :N]
        return (dinv[:, None] * (agg + xs)) @ W + b

    xa = gcn(x, s0, d0, indeg0, gcn1_W, gcn1_b)
    xb = gcn(x1, s1, d1, indeg1, gcn2_W, gcn2_b)
    x0 = jax.nn.relu(xa) + jax.nn.relu(xb)

    def sage(xin, src, dst, indeg, Wn, bn, Ws):
        p = xin @ Wn
        parts = _rowagg128(p, src, dst, zeros_rows)
        agg = (parts[0, :N] + parts[1, :N]) / jnp.maximum(indeg, 1.0)[:, None]
        return agg + bn + xin @ Ws

    xc_s = sage(x0, s0, d0, indeg0, sage1_Wn, sage1_b, sage1_Ws)

    def gat(xin, src, dst, W, Wb, a, ab):
        h = xin @ W + Wb
        hd = h @ a[: h.shape[1], 0] + ab[0]
        hs = h @ a[h.shape[1]:, 0]
        h128 = jnp.concatenate([h, jnp.zeros((N, 64), jnp.float32)], axis=1)
        nparts, dparts = _gat_kernel(h128, hd, hs, src, dst, zeros_rows,
                                     zeros_1d)
        num = nparts[0, :N, :64] + nparts[1, :N, :64]
        dparts = dparts.reshape(NC, NPAD)
        denom = dparts[0, :N] + dparts[1, :N]
        return num / jnp.where(denom > 0, denom, 1.0)[:, None]

    xc = jax.nn.relu(gat(xc_s, s0, d0, gat1_W, gat1_Wb, gat1_a, gat1_ab))

    def mlp(z):
        z = z @ mlp_W1 + mlp_b1
        mu = jnp.mean(z, axis=-1, keepdims=True)
        var = jnp.var(z, axis=-1, keepdims=True)
        z = (z - mu) / jnp.sqrt(var + 1e-5) * ln_g + ln_b
        return z @ mlp_W2 + mlp_b2

    xc = mlp(xc)

    x1_0b = jax.nn.relu(xb) + xb
    xd_s = sage(x1_0b, s1, d1, indeg1, sage2_Wn, sage2_b, sage2_Ws)
    xd = jax.nn.relu(gat(xd_s, s1, d1, gat2_W, gat2_Wb, gat2_a, gat2_ab))
    xd = mlp(xd)

    cat = jnp.concatenate([xc, xd], axis=0)
    total = pl.pallas_call(
        _mean_body,
        out_shape=jax.ShapeDtypeStruct((1, 1), jnp.float32),
    )(cat)
    return (total / (cat.shape[0] * cat.shape[1])).reshape(1, 1)
